# baseline (speedup 1.0000x reference)
import functools

import jax
import jax.numpy as jnp
from jax import lax
from jax.experimental import pallas as pl
from jax.experimental.pallas import tpu as pltpu
from jax.experimental.pallas import tpu_sc as plsc

_BATCH = 8192
_HIDDEN = 2048
_NUM_OPTIONS = 8
_NUM_ACTIONS = 512
_BT = 128
_NB = 72
_P = _NB * _BT

_NW = 32
_RPW = _P // _NW
_CH = 16
_NCH = _RPW // _CH
_NBUF = 3


@functools.partial(
    pl.kernel,
    out_type=jax.ShapeDtypeStruct((_P, _HIDDEN), jnp.float32),
    mesh=plsc.VectorSubcoreMesh(core_axis_name="c", subcore_axis_name="s"),
    scratch_types=[
        pltpu.VMEM((_RPW,), jnp.int32),
    ] + [pltpu.VMEM((_CH, _HIDDEN), jnp.float32) for _ in range(_NBUF)]
      + [pltpu.SemaphoreType.DMA for _ in range(2 * _NBUF)],
)
def _sc_gather(states_hbm, idx_hbm, out_hbm, idx_v, *bufs_sems):
    bufs = bufs_sems[:_NBUF]
    rsems = bufs_sems[_NBUF:2 * _NBUF]
    wsems = bufs_sems[2 * _NBUF:]
    wid = lax.axis_index("s") * 2 + lax.axis_index("c")
    base = wid * _RPW
    pltpu.sync_copy(idx_hbm.at[pl.ds(base, _RPW)], idx_v)

    rd, wr = {}, {}

    def start_read(c):
        rd[c] = pltpu.async_copy(
            states_hbm.at[idx_v.at[pl.ds(c * _CH, _CH)]],
            bufs[c % _NBUF], rsems[c % _NBUF])

    def start_write(c):
        wr[c] = pltpu.async_copy(
            bufs[c % _NBUF], out_hbm.at[pl.ds(base + c * _CH, _CH)],
            wsems[c % _NBUF])

    for c in range(min(_NBUF, _NCH)):
        start_read(c)
    for c in range(_NCH):
        rd[c].wait()
        start_write(c)
        if c + _NBUF < _NCH:
            wr[c].wait()
            start_read(c + _NBUF)
    for c in range(max(0, _NCH - _NBUF), _NCH):
        wr[c].wait()


def _moe_body(be_ref, x_ref, w_ref, b_ref, a_ref, lp_ref, ent_ref):
    del be_ref
    x = x_ref[...].astype(jnp.bfloat16)
    w = w_ref[0]
    logits = jax.lax.dot_general(
        x, w, (((1,), (0,)), ((), ())),
        preferred_element_type=jnp.float32,
    ) + b_ref[0, 0]
    m = jnp.max(logits, axis=-1, keepdims=True)
    s = logits - m
    es = jnp.exp(s)
    denom = jnp.sum(es, axis=-1, keepdims=True)
    logp = s - jnp.log(denom)
    probs = es / denom
    ent = -jnp.sum(probs * logp, axis=-1)
    a = a_ref[0, 0]
    sel = jax.lax.broadcasted_iota(jnp.int32, (_BT, _NUM_ACTIONS), 1) == a[:, None]
    lp_sel = jnp.sum(jnp.where(sel, logp, 0.0), axis=-1)
    lp_ref[0, 0] = lp_sel
    ent_ref[0, 0] = ent


@jax.jit
def kernel(states, options, actions_old, W, b):
    opts = options.astype(jnp.int32)
    acts = actions_old.astype(jnp.int32)

    onehot = (opts[:, None] == jnp.arange(_NUM_OPTIONS, dtype=jnp.int32)[None, :])
    counts = jnp.sum(onehot, axis=0)
    blocks_per = (counts + _BT - 1) // _BT
    blk_end = jnp.cumsum(blocks_per)
    blk_start = blk_end - blocks_per
    padded_off = blk_start * _BT
    rank = jnp.cumsum(onehot, axis=0) - 1
    my_rank = jnp.take_along_axis(rank, opts[:, None], axis=1)[:, 0]
    pos = padded_off[opts] + my_rank
    gather_idx = jnp.zeros((_P,), jnp.int32).at[pos].set(
        jnp.arange(_BATCH, dtype=jnp.int32))
    block_expert = jnp.minimum(
        jnp.sum(jnp.arange(_NB, dtype=jnp.int32)[:, None] >= blk_end[None, :],
                axis=1),
        _NUM_OPTIONS - 1).astype(jnp.int32)

    W_bf = W.astype(jnp.bfloat16)
    x_sorted = _sc_gather(states, gather_idx)
    a_sorted = jnp.take(acts, gather_idx).reshape(_NB, 1, _BT)

    grid_spec = pltpu.PrefetchScalarGridSpec(
        num_scalar_prefetch=1,
        grid=(_NB,),
        in_specs=[
            pl.BlockSpec((_BT, _HIDDEN), lambda i, be: (i, 0)),
            pl.BlockSpec((1, _HIDDEN, _NUM_ACTIONS), lambda i, be: (be[i], 0, 0)),
            pl.BlockSpec((1, 1, _NUM_ACTIONS), lambda i, be: (be[i], 0, 0)),
            pl.BlockSpec((1, 1, _BT), lambda i, be: (i, 0, 0)),
        ],
        out_specs=[
            pl.BlockSpec((1, 1, _BT), lambda i, be: (i, 0, 0)),
            pl.BlockSpec((1, 1, _BT), lambda i, be: (i, 0, 0)),
        ],
    )
    lp_s, ent_s = pl.pallas_call(
        _moe_body,
        grid_spec=grid_spec,
        out_shape=[
            jax.ShapeDtypeStruct((_NB, 1, _BT), jnp.float32),
            jax.ShapeDtypeStruct((_NB, 1, _BT), jnp.float32),
        ],
    )(block_expert, x_sorted, W_bf, b.reshape(_NUM_OPTIONS, 1, _NUM_ACTIONS),
      a_sorted)

    log_probs = lp_s.reshape(-1)[pos]
    entropy = ent_s.reshape(-1)[pos]
    return (log_probs, entropy)


# speedup vs baseline: 1.0930x; 1.0930x over previous
import functools

import jax
import jax.numpy as jnp
from jax import lax
from jax.experimental import pallas as pl
from jax.experimental.pallas import tpu as pltpu
from jax.experimental.pallas import tpu_sc as plsc

_BATCH = 8192
_HIDDEN = 2048
_NUM_OPTIONS = 8
_NUM_ACTIONS = 512
_BT = 128
_NB = 72
_P = _NB * _BT
_NCHUNK = 4
_CB = _NB // _NCHUNK
_CROWS = _CB * _BT

_NW = 32
_RPW = _CROWS // _NW
_CH = 24
_NCH = _RPW // _CH
_NBUF = 2


def _make_sc_gather(chunk):
    base_slot = chunk * _CROWS

    @functools.partial(
        pl.kernel,
        out_type=jax.ShapeDtypeStruct((_CROWS, _HIDDEN), jnp.float32),
        mesh=plsc.VectorSubcoreMesh(core_axis_name="c", subcore_axis_name="s"),
        scratch_types=[
            pltpu.VMEM((_RPW,), jnp.int32),
        ] + [pltpu.VMEM((_CH, _HIDDEN), jnp.float32) for _ in range(_NBUF)]
          + [pltpu.SemaphoreType.DMA for _ in range(2 * _NBUF)],
    )
    def _sc_gather(states_hbm, idx_hbm, out_hbm, idx_v, *bufs_sems):
        bufs = bufs_sems[:_NBUF]
        rsems = bufs_sems[_NBUF:2 * _NBUF]
        wsems = bufs_sems[2 * _NBUF:]
        wid = lax.axis_index("s") * 2 + lax.axis_index("c")
        base = wid * _RPW
        pltpu.sync_copy(idx_hbm.at[pl.ds(base_slot + base, _RPW)], idx_v)

        rd, wr = {}, {}

        def start_read(c):
            rd[c] = pltpu.async_copy(
                states_hbm.at[idx_v.at[pl.ds(c * _CH, _CH)]],
                bufs[c % _NBUF], rsems[c % _NBUF])

        def start_write(c):
            wr[c] = pltpu.async_copy(
                bufs[c % _NBUF], out_hbm.at[pl.ds(base + c * _CH, _CH)],
                wsems[c % _NBUF])

        for c in range(min(_NBUF, _NCH)):
            start_read(c)
        for c in range(_NCH):
            rd[c].wait()
            start_write(c)
            if c + _NBUF < _NCH:
                wr[c].wait()
                start_read(c + _NBUF)
        for c in range(max(0, _NCH - _NBUF), _NCH):
            wr[c].wait()

    return _sc_gather


_SC_GATHERS = [_make_sc_gather(k) for k in range(_NCHUNK)]


def _moe_body(be_ref, x_ref, w_ref, b_ref, a_ref, lp_ref, ent_ref):
    del be_ref
    x = x_ref[...].astype(jnp.bfloat16)
    w = w_ref[0]
    logits = jax.lax.dot_general(
        x, w, (((1,), (0,)), ((), ())),
        preferred_element_type=jnp.float32,
    ) + b_ref[0, 0]
    m = jnp.max(logits, axis=-1, keepdims=True)
    s = logits - m
    es = jnp.exp(s)
    denom = jnp.sum(es, axis=-1, keepdims=True)
    ld = jnp.log(denom)
    ssum = jnp.sum(es * s, axis=-1, keepdims=True)
    ent = (ld - ssum / denom)[:, 0]
    a = a_ref[0, 0]
    sel = jax.lax.broadcasted_iota(jnp.int32, (_BT, _NUM_ACTIONS), 1) == a[:, None]
    lp_sel = jnp.sum(jnp.where(sel, s, 0.0), axis=-1) - ld[:, 0]
    lp_ref[0, 0] = lp_sel
    ent_ref[0, 0] = ent


def _moe_chunk(block_expert_c, x_c, W_bf, b3, a_c):
    grid_spec = pltpu.PrefetchScalarGridSpec(
        num_scalar_prefetch=1,
        grid=(_CB,),
        in_specs=[
            pl.BlockSpec((_BT, _HIDDEN), lambda i, be: (i, 0)),
            pl.BlockSpec((1, _HIDDEN, _NUM_ACTIONS), lambda i, be: (be[i], 0, 0)),
            pl.BlockSpec((1, 1, _NUM_ACTIONS), lambda i, be: (be[i], 0, 0)),
            pl.BlockSpec((1, 1, _BT), lambda i, be: (i, 0, 0)),
        ],
        out_specs=[
            pl.BlockSpec((1, 1, _BT), lambda i, be: (i, 0, 0)),
            pl.BlockSpec((1, 1, _BT), lambda i, be: (i, 0, 0)),
        ],
    )
    return pl.pallas_call(
        _moe_body,
        grid_spec=grid_spec,
        out_shape=[
            jax.ShapeDtypeStruct((_CB, 1, _BT), jnp.float32),
            jax.ShapeDtypeStruct((_CB, 1, _BT), jnp.float32),
        ],
    )(block_expert_c, x_c, W_bf, b3, a_c)


@jax.jit
def kernel(states, options, actions_old, W, b):
    opts = options.astype(jnp.int32)
    acts = actions_old.astype(jnp.int32)

    onehot = (opts[:, None] == jnp.arange(_NUM_OPTIONS, dtype=jnp.int32)[None, :])
    counts = jnp.sum(onehot, axis=0)
    blocks_per = (counts + _BT - 1) // _BT
    blk_end = jnp.cumsum(blocks_per)
    blk_start = blk_end - blocks_per
    padded_off = blk_start * _BT
    rank = jnp.cumsum(onehot, axis=0) - 1
    my_rank = jnp.take_along_axis(rank, opts[:, None], axis=1)[:, 0]
    pos = padded_off[opts] + my_rank
    gather_idx = jnp.zeros((_P,), jnp.int32).at[pos].set(
        jnp.arange(_BATCH, dtype=jnp.int32))
    block_expert = jnp.minimum(
        jnp.sum(jnp.arange(_NB, dtype=jnp.int32)[:, None] >= blk_end[None, :],
                axis=1),
        _NUM_OPTIONS - 1).astype(jnp.int32)

    W_bf = W.astype(jnp.bfloat16)
    b3 = b.reshape(_NUM_OPTIONS, 1, _NUM_ACTIONS)
    a_sorted = jnp.take(acts, gather_idx).reshape(_NB, 1, _BT)

    lp_chunks, ent_chunks = [], []
    for k in range(_NCHUNK):
        x_c = _SC_GATHERS[k](states, gather_idx)
        lp_c, ent_c = _moe_chunk(
            block_expert[k * _CB:(k + 1) * _CB], x_c, W_bf, b3,
            a_sorted[k * _CB:(k + 1) * _CB])
        lp_chunks.append(lp_c)
        ent_chunks.append(ent_c)
    lp_s = jnp.concatenate(lp_chunks, axis=0)
    ent_s = jnp.concatenate(ent_chunks, axis=0)

    log_probs = lp_s.reshape(-1)[pos]
    entropy = ent_s.reshape(-1)[pos]
    return (log_probs, entropy)
